# baseline (speedup 1.0000x reference)
import functools

import jax
import jax.numpy as jnp
from jax.experimental import pallas as pl
from jax.experimental.pallas import tpu as pltpu

B, N, H, P, K, CIN, COUT = 1, 10000, 32, 3, 15, 128, 128
KP_EXTENT = 1.2
TN = 400


def _kpconv_kernel(q_ref, nb_ref, x_ref, w_ref, kp_ref, out_ref):
    q = q_ref[...]
    nb = nb_ref[...]
    kp = kp_ref[...]

    diff = nb - q[:, None, :]
    sq_d = jnp.zeros((TN, H, K), dtype=jnp.float32)
    for p in range(P):
        d_p = diff[:, :, p][:, :, None]
        k_p = kp[p, :][None, None, :]
        t = d_p - k_p
        sq_d = sq_d + t * t
    w3 = jnp.maximum(1.0 - jnp.sqrt(sq_d) * (1.0 / KP_EXTENT), 0.0)

    x = x_ref[...]
    acc = jnp.zeros((TN, COUT), dtype=jnp.float32)
    for k in range(K):
        wk = w3[:, :, k][:, :, None]
        y_k = jnp.sum(wk * x, axis=1)
        acc = acc + jnp.dot(y_k, w_ref[k], preferred_element_type=jnp.float32)
    out_ref[...] = acc


@functools.partial(jax.jit, static_argnames=())
def kernel(q_pts, s_pts, neighbors, neighb_inds, x, weights, kernel_points):
    del s_pts, neighb_inds
    q = q_pts.reshape(N, P)
    nb = neighbors.reshape(N, H, P)
    xf = x.reshape(N, H, CIN)
    kp_t = kernel_points.T

    grid = (N // TN,)
    out = pl.pallas_call(
        _kpconv_kernel,
        grid=grid,
        in_specs=[
            pl.BlockSpec((TN, P), lambda i: (i, 0)),
            pl.BlockSpec((TN, H, P), lambda i: (i, 0, 0)),
            pl.BlockSpec((TN, H, CIN), lambda i: (i, 0, 0)),
            pl.BlockSpec((K, CIN, COUT), lambda i: (0, 0, 0)),
            pl.BlockSpec((P, K), lambda i: (0, 0)),
        ],
        out_specs=pl.BlockSpec((TN, COUT), lambda i: (i, 0)),
        out_shape=jax.ShapeDtypeStruct((N, COUT), jnp.float32),
        compiler_params=pltpu.CompilerParams(
            dimension_semantics=("arbitrary",),
        ),
    )(q, nb, xf, weights, kp_t)
    return out.reshape(B, N, COUT)


# speedup vs baseline: 5.1095x; 5.1095x over previous
import functools

import jax
import jax.numpy as jnp
import numpy as np
from jax.experimental import pallas as pl
from jax.experimental.pallas import tpu as pltpu

B, N, H, P, K, CIN, COUT = 1, 10000, 32, 3, 15, 128, 128
K16 = 16
KP_EXTENT = 1.2
TN = 80
G = TN // 8
NSTEP = N // TN
LW = G * H
LA = G * 8 * H
GSPLIT = 8


def _kpconv_kernel(d_ref, x_ref, w_ref, kpc_ref, rbig_ref, mask_ref,
                   out_ref, ys_ref):
    sq = jnp.zeros((K16 * 8, LW), dtype=jnp.float32)
    for c in range(P):
        d_c = d_ref[c, 0]
        dt = jnp.broadcast_to(d_c[None], (K16, 8, LW)).reshape(K16 * 8, LW)
        t = dt - kpc_ref[c]
        sq = sq + t * t
    wfull = jnp.maximum(1.0 - jnp.sqrt(sq) * (1.0 / KP_EXTENT), 0.0)
    wfull = wfull.astype(jnp.bfloat16)

    s = GSPLIT * H
    a_lo = jnp.dot(wfull[:, :s], rbig_ref[:s, :GSPLIT * 256],
                   preferred_element_type=jnp.float32)
    a_hi = jnp.dot(wfull[:, s:], rbig_ref[s:, GSPLIT * 256:],
                   preferred_element_type=jnp.float32)
    afull = jnp.concatenate([a_lo, a_hi], axis=1).astype(jnp.bfloat16)
    afull = afull * mask_ref[...]

    for g in range(G):
        x_g = x_ref[g * 8:(g + 1) * 8].reshape(8 * H, CIN).astype(jnp.bfloat16)
        ys_ref[g] = jnp.dot(afull[:, g * 256:(g + 1) * 256], x_g,
                            preferred_element_type=jnp.float32)

    ycat = jnp.concatenate(
        [ys_ref[:, k * 8:(k + 1) * 8, :].reshape(TN, CIN).astype(jnp.bfloat16)
         for k in range(K)], axis=1)
    out_ref[...] = jnp.dot(ycat, w_ref[...],
                           preferred_element_type=jnp.float32)


@functools.partial(jax.jit, static_argnames=())
def kernel(q_pts, s_pts, neighbors, neighb_inds, x, weights, kernel_points):
    del s_pts, neighb_inds
    xf = x.reshape(N, H, CIN)
    wstack = weights.reshape(K * CIN, COUT).astype(jnp.bfloat16)

    diff = (neighbors - q_pts[:, :, None, :]).reshape(NSTEP, G, 8, H, P)
    dT = diff.transpose(4, 0, 2, 1, 3).reshape(P, NSTEP, 8, LW)

    kp_pad = jnp.concatenate(
        [kernel_points, jnp.full((K16 - K, P), 1e3, jnp.float32)], axis=0)
    kpcols = jnp.transpose(kp_pad)[:, np.repeat(np.arange(K16), 8), None]

    rr = np.arange(K16 * 8)[:, None]
    ca = np.arange(LA)[None, :]
    maskbig = jnp.asarray(((ca // H) % 8 == rr % 8).astype(np.float32),
                          jnp.bfloat16)
    rw = np.arange(LW)[:, None]
    rbig = jnp.asarray(
        ((ca // 256 == rw // H) & (ca % H == rw % H)).astype(np.float32),
        jnp.bfloat16)

    grid = (NSTEP,)
    out = pl.pallas_call(
        _kpconv_kernel,
        grid=grid,
        in_specs=[
            pl.BlockSpec((P, 1, 8, LW), lambda i: (0, i, 0, 0)),
            pl.BlockSpec((TN, H, CIN), lambda i: (i, 0, 0)),
            pl.BlockSpec((K * CIN, COUT), lambda i: (0, 0)),
            pl.BlockSpec((P, K16 * 8, 1), lambda i: (0, 0, 0)),
            pl.BlockSpec((LW, LA), lambda i: (0, 0)),
            pl.BlockSpec((K16 * 8, LA), lambda i: (0, 0)),
        ],
        out_specs=pl.BlockSpec((TN, COUT), lambda i: (i, 0)),
        out_shape=jax.ShapeDtypeStruct((N, COUT), jnp.float32),
        scratch_shapes=[pltpu.VMEM((G, K16 * 8, CIN), jnp.float32)],
        compiler_params=pltpu.CompilerParams(
            dimension_semantics=("arbitrary",),
        ),
    )(dT, xf, wstack, kpcols, rbig, maskbig)
    return out.reshape(B, N, COUT)


# speedup vs baseline: 7.0805x; 1.3857x over previous
import functools

import jax
import jax.numpy as jnp
import numpy as np
from jax.experimental import pallas as pl
from jax.experimental.pallas import tpu as pltpu

B, N, H, P, K, CIN, COUT = 1, 10000, 32, 3, 15, 128, 128
K16 = 16
KP_EXTENT = 1.2
TN = 400
G = TN // 8
NSTEP = N // TN
LW = G * H


def _kpconv_kernel(d_ref, x_ref, w_ref, kpc_ref, r32_ref, mask_ref,
                   out_ref, ys_ref, ws_ref):
    sq = jnp.zeros((K16 * 8, LW), dtype=jnp.float32)
    for c in range(P):
        d_c = d_ref[c, 0]
        dt = jnp.broadcast_to(d_c[None], (K16, 8, LW)).reshape(K16 * 8, LW)
        t = dt - kpc_ref[c]
        sq = sq + t * t
    ws_ref[...] = jnp.maximum(1.0 - jnp.sqrt(sq) * (1.0 / KP_EXTENT), 0.0)

    mask = mask_ref[...]
    r32 = r32_ref[...]
    for g in range(G):
        w_g = ws_ref[:, g * H:(g + 1) * H].astype(jnp.bfloat16)
        a_g = jnp.dot(w_g, r32, preferred_element_type=jnp.float32)
        a_g = a_g.astype(jnp.bfloat16) * mask
        x_g = x_ref[g * 8:(g + 1) * 8].reshape(8 * H, CIN).astype(jnp.bfloat16)
        ys_ref[g] = jnp.dot(a_g, x_g, preferred_element_type=jnp.float32)

    acc = jnp.zeros((TN, COUT), dtype=jnp.float32)
    for k in range(K):
        y_k = ys_ref[:, k * 8:(k + 1) * 8, :].reshape(TN, CIN).astype(jnp.bfloat16)
        acc = acc + jnp.dot(y_k, w_ref[k], preferred_element_type=jnp.float32)
    out_ref[...] = acc


@functools.partial(jax.jit, static_argnames=())
def kernel(q_pts, s_pts, neighbors, neighb_inds, x, weights, kernel_points):
    del s_pts, neighb_inds
    xf = x.reshape(N, H, CIN)
    wts = weights.astype(jnp.bfloat16)

    diff = (neighbors - q_pts[:, :, None, :]).reshape(NSTEP, G, 8, H, P)
    dT = diff.transpose(4, 0, 2, 1, 3).reshape(P, NSTEP, 8, LW)

    kp_pad = jnp.concatenate(
        [kernel_points, jnp.full((K16 - K, P), 1e3, jnp.float32)], axis=0)
    kpcols = jnp.transpose(kp_pad)[:, np.repeat(np.arange(K16), 8), None]

    rr = np.arange(K16 * 8)[:, None]
    cc = np.arange(8 * H)[None, :]
    mask = jnp.asarray((cc // H == rr % 8).astype(np.float32), jnp.bfloat16)
    r32 = jnp.asarray((cc % H == np.arange(H)[:, None]).astype(np.float32),
                      jnp.bfloat16)

    grid = (NSTEP,)
    out = pl.pallas_call(
        _kpconv_kernel,
        grid=grid,
        in_specs=[
            pl.BlockSpec((P, 1, 8, LW), lambda i: (0, i, 0, 0)),
            pl.BlockSpec((TN, H, CIN), lambda i: (i, 0, 0)),
            pl.BlockSpec((K, CIN, COUT), lambda i: (0, 0, 0)),
            pl.BlockSpec((P, K16 * 8, 1), lambda i: (0, 0, 0)),
            pl.BlockSpec((H, 8 * H), lambda i: (0, 0)),
            pl.BlockSpec((K16 * 8, 8 * H), lambda i: (0, 0)),
        ],
        out_specs=pl.BlockSpec((TN, COUT), lambda i: (i, 0)),
        out_shape=jax.ShapeDtypeStruct((N, COUT), jnp.float32),
        scratch_shapes=[pltpu.VMEM((G, K16 * 8, CIN), jnp.float32),
                        pltpu.VMEM((K16 * 8, LW), jnp.float32)],
        compiler_params=pltpu.CompilerParams(
            dimension_semantics=("arbitrary",),
        ),
    )(dT, xf, wts, kpcols, r32, mask)
    return out.reshape(B, N, COUT)
